# Initial kernel scaffold; baseline (speedup 1.0000x reference)
import jax
import jax.numpy as jnp
from jax.experimental import pallas as pl


def kernel(x, edge_index, Wm, bm, Ws, bs, gamma, beta):
    raise NotImplementedError("write your pallas kernel here")


# speedup vs baseline: 2.6753x; 2.6753x over previous
import functools

import jax
import jax.numpy as jnp
from jax import lax
from jax.experimental import pallas as pl
from jax.experimental.pallas import tpu as pltpu
from jax.experimental.pallas import tpu_sc as plsc

N = 10000
E = 320000
D = 128

NS = 16
NW = NS
CHUNK = 128
GRP = 4
NGRP = -(-E // (NW * CHUNK * GRP))
NCHUNK = NGRP * GRP
E_PAD = NW * NCHUNK * CHUNK
PADC = float(E_PAD - E)
RPT = 632
RPT_LAST = N - (NS - 1) * RPT
NTA = RPT - (RPT // CHUNK) * CHUNK
NTB = RPT_LAST - (RPT // CHUNK) * CHUNK


def _fill_block(ref, val):
    def _row(r, carry):
        for k in range(D // 16):
            ref[r, pl.ds(k * 16, 16)] = jnp.full((16,), val, jnp.float32)
        return carry
    lax.fori_loop(0, CHUNK, _row, 0)


def _zero_acc_slice(s, buf, acc_sh):
    base = s * RPT
    for k in range(RPT // CHUNK):
        pltpu.sync_copy(buf, acc_sh.at[pl.ds(base + k * CHUNK, CHUNK)])
    toff = base + (RPT // CHUNK) * CHUNK

    @pl.when(s < NS - 1)
    def _():
        pltpu.sync_copy(buf.at[pl.ds(0, NTA)], acc_sh.at[pl.ds(toff, NTA)])

    @pl.when(s == NS - 1)
    def _():
        pltpu.sync_copy(buf.at[pl.ds(0, NTB)], acc_sh.at[pl.ds(toff, NTB)])


def _write_acc_slice(s, buf, acc_sh, out_hbm):
    base = s * RPT
    for k in range(RPT // CHUNK):
        off = base + k * CHUNK
        pltpu.sync_copy(acc_sh.at[pl.ds(off, CHUNK)], buf)
        pltpu.sync_copy(buf, out_hbm.at[pl.ds(off, CHUNK)])
    toff = base + (RPT // CHUNK) * CHUNK

    @pl.when(s < NS - 1)
    def _():
        pltpu.sync_copy(acc_sh.at[pl.ds(toff, NTA)], buf.at[pl.ds(0, NTA)])
        pltpu.sync_copy(buf.at[pl.ds(0, NTA)], out_hbm.at[pl.ds(toff, NTA)])

    @pl.when(s == NS - 1)
    def _():
        pltpu.sync_copy(acc_sh.at[pl.ds(toff, NTB)], buf.at[pl.ds(0, NTB)])
        pltpu.sync_copy(buf.at[pl.ds(0, NTB)], out_hbm.at[pl.ds(toff, NTB)])


@functools.cache
def _make_sc_agg():
    mesh = plsc.VectorSubcoreMesh(core_axis_name="c", subcore_axis_name="s",
                                  num_cores=1)
    return functools.partial(
        pl.kernel,
        out_type=jax.ShapeDtypeStruct((N, D), jnp.float32),
        mesh=mesh,
        scratch_types=[
            pltpu.VMEM((GRP, CHUNK), jnp.int32),
            pltpu.VMEM((GRP, CHUNK), jnp.int32),
            pltpu.VMEM((CHUNK, D), jnp.float32),
            pltpu.VMEM((CHUNK, D), jnp.float32),
            pltpu.VMEM_SHARED((N, D), jnp.float32),
            pltpu.SemaphoreType.DMA,
            pltpu.SemaphoreType.DMA,
        ],
    )(_sc_agg_body)


def _sc_agg_body(x_hbm, src_hbm, dst_hbm, agg_out,
                 src_v, dst_v, rows_a, rows_b, acc_sh, sem_a, sem_b):
    s = lax.axis_index("s")
    _fill_block(rows_a, 0.0)
    _zero_acc_slice(s, rows_a, acc_sh)
    plsc.subcore_barrier()

    def _group(g, carry):
        gb = s * NGRP + g
        pltpu.sync_copy(src_hbm.at[gb], src_v)
        pltpu.sync_copy(dst_hbm.at[gb], dst_v)
        for b in range(0, GRP, 2):
            cpa = pltpu.async_copy(x_hbm.at[src_v.at[b]], rows_a, sem_a)
            cpb = pltpu.async_copy(x_hbm.at[src_v.at[b + 1]], rows_b, sem_b)
            cpa.wait()
            pltpu.sync_copy(rows_a, acc_sh.at[dst_v.at[b]], add=True)
            cpb.wait()
            pltpu.sync_copy(rows_b, acc_sh.at[dst_v.at[b + 1]], add=True)
        return carry
    lax.fori_loop(0, NGRP, _group, 0)

    plsc.subcore_barrier()
    _write_acc_slice(s, rows_a, acc_sh, agg_out)


@functools.cache
def _make_sc_deg():
    mesh = plsc.VectorSubcoreMesh(core_axis_name="c", subcore_axis_name="s",
                                  num_cores=1)
    return functools.partial(
        pl.kernel,
        out_type=jax.ShapeDtypeStruct((N, D), jnp.float32),
        mesh=mesh,
        scratch_types=[
            pltpu.VMEM((GRP, CHUNK), jnp.int32),
            pltpu.VMEM((CHUNK, D), jnp.float32),
            pltpu.VMEM_SHARED((N, D), jnp.float32),
        ],
    )(_sc_deg_body)


def _sc_deg_body(dst_hbm, deg_out, dst_v, ones_v, deg_sh):
    s = lax.axis_index("s")
    _fill_block(ones_v, 0.0)
    _zero_acc_slice(s, ones_v, deg_sh)
    _fill_block(ones_v, 1.0)
    plsc.subcore_barrier()

    def _group(g, carry):
        gb = s * NGRP + g
        pltpu.sync_copy(dst_hbm.at[gb], dst_v)
        for b in range(GRP):
            pltpu.sync_copy(ones_v, deg_sh.at[dst_v.at[b]], add=True)
        return carry
    lax.fori_loop(0, NGRP, _group, 0)

    plsc.subcore_barrier()
    _write_acc_slice(s, ones_v, deg_sh, deg_out)


BLK = 2000


def _tc_body(x_ref, a_ref, d_ref, wm_ref, bm_ref, ws_ref, bs_ref,
             g_ref, b_ref, o_ref):
    x = x_ref[...]
    a = a_ref[...]
    deg = d_ref[:, 0:1]
    pid = pl.program_id(0)
    row0 = (lax.broadcasted_iota(jnp.int32, (BLK, 1), 0) == 0).astype(jnp.float32)
    row0 = row0 * (pid == 0).astype(jnp.float32)
    a = a - row0 * (PADC * x[0:1, :])
    deg = deg - row0 * PADC
    h = lax.dot_general(a, wm_ref[...], (((1,), (1,)), ((), ())),
                        precision=lax.Precision.HIGHEST,
                        preferred_element_type=jnp.float32)
    agg = (h + deg * bm_ref[...]) / jnp.maximum(deg, 1.0)
    o = lax.dot_general(x, ws_ref[...], (((1,), (1,)), ((), ())),
                        precision=lax.Precision.HIGHEST,
                        preferred_element_type=jnp.float32)
    o = o + bs_ref[...] + agg
    u = x + 0.5 * o * (1.0 + lax.erf(o * 0.7071067811865475))
    mu = jnp.mean(u, axis=1, keepdims=True)
    uc = u - mu
    var = jnp.mean(uc * uc, axis=1, keepdims=True)
    o_ref[...] = uc * lax.rsqrt(var + 1e-5) * g_ref[...] + b_ref[...]


def kernel(x, edge_index, Wm, bm, Ws, bs, gamma, beta):
    src = edge_index[0].astype(jnp.int32)
    dst = edge_index[1].astype(jnp.int32)
    pad = E_PAD - E
    src = jnp.concatenate([src, jnp.zeros((pad,), jnp.int32)])
    dst = jnp.concatenate([dst, jnp.zeros((pad,), jnp.int32)])
    src2 = src.reshape(NW * NGRP, GRP, CHUNK)
    dst2 = dst.reshape(NW * NGRP, GRP, CHUNK)

    agg = _make_sc_agg()(x, src2, dst2)
    deg = _make_sc_deg()(dst2)

    out = pl.pallas_call(
        _tc_body,
        grid=(N // BLK,),
        in_specs=[
            pl.BlockSpec((BLK, D), lambda i: (i, 0)),
            pl.BlockSpec((BLK, D), lambda i: (i, 0)),
            pl.BlockSpec((BLK, D), lambda i: (i, 0)),
            pl.BlockSpec((D, D), lambda i: (0, 0)),
            pl.BlockSpec((1, D), lambda i: (0, 0)),
            pl.BlockSpec((D, D), lambda i: (0, 0)),
            pl.BlockSpec((1, D), lambda i: (0, 0)),
            pl.BlockSpec((1, D), lambda i: (0, 0)),
            pl.BlockSpec((1, D), lambda i: (0, 0)),
        ],
        out_specs=pl.BlockSpec((BLK, D), lambda i: (i, 0)),
        out_shape=jax.ShapeDtypeStruct((N, D), jnp.float32),
    )(x, agg, deg, Wm, bm.reshape(1, D), Ws, bs.reshape(1, D),
      gamma.reshape(1, D), beta.reshape(1, D))
    return out


# speedup vs baseline: 3.3347x; 1.2465x over previous
import functools

import jax
import jax.numpy as jnp
from jax import lax
from jax.experimental import pallas as pl
from jax.experimental.pallas import tpu as pltpu
from jax.experimental.pallas import tpu_sc as plsc

N = 10000
E = 320000
D = 128

NC = 2
NS = 16
NW = NC * NS
CHUNK = 128
GRP = 4
NGRP = -(-E // (NW * CHUNK * GRP))
NCHUNK = NGRP * GRP
E_PAD = NW * NCHUNK * CHUNK
PADC = float(E_PAD - E)
RPT = 632
RPT_LAST = N - (NS - 1) * RPT
NTA = RPT - (RPT // CHUNK) * CHUNK
NTB = RPT_LAST - (RPT // CHUNK) * CHUNK


def _fill_block(ref, val):
    def _row(r, carry):
        for k in range(D // 16):
            ref[r, pl.ds(k * 16, 16)] = jnp.full((16,), val, jnp.float32)
        return carry
    lax.fori_loop(0, CHUNK, _row, 0)


def _zero_acc_slice(s, buf, acc_sh):
    base = s * RPT
    for k in range(RPT // CHUNK):
        pltpu.sync_copy(buf, acc_sh.at[pl.ds(base + k * CHUNK, CHUNK)])
    toff = base + (RPT // CHUNK) * CHUNK

    @pl.when(s < NS - 1)
    def _():
        pltpu.sync_copy(buf.at[pl.ds(0, NTA)], acc_sh.at[pl.ds(toff, NTA)])

    @pl.when(s == NS - 1)
    def _():
        pltpu.sync_copy(buf.at[pl.ds(0, NTB)], acc_sh.at[pl.ds(toff, NTB)])


def _write_acc_slice(c, s, buf, acc_sh, out_hbm):
    base = s * RPT
    hb = c * N
    for k in range(RPT // CHUNK):
        off = base + k * CHUNK
        pltpu.sync_copy(acc_sh.at[pl.ds(off, CHUNK)], buf)
        pltpu.sync_copy(buf, out_hbm.at[pl.ds(hb + off, CHUNK)])
    toff = base + (RPT // CHUNK) * CHUNK

    @pl.when(s < NS - 1)
    def _():
        pltpu.sync_copy(acc_sh.at[pl.ds(toff, NTA)], buf.at[pl.ds(0, NTA)])
        pltpu.sync_copy(buf.at[pl.ds(0, NTA)], out_hbm.at[pl.ds(hb + toff, NTA)])

    @pl.when(s == NS - 1)
    def _():
        pltpu.sync_copy(acc_sh.at[pl.ds(toff, NTB)], buf.at[pl.ds(0, NTB)])
        pltpu.sync_copy(buf.at[pl.ds(0, NTB)], out_hbm.at[pl.ds(hb + toff, NTB)])


@functools.cache
def _make_sc_agg():
    mesh = plsc.VectorSubcoreMesh(core_axis_name="c", subcore_axis_name="s",
                                  num_cores=NC)
    return functools.partial(
        pl.kernel,
        out_type=jax.ShapeDtypeStruct((NC * N, D), jnp.float32),
        mesh=mesh,
        scratch_types=[
            pltpu.VMEM((GRP, CHUNK), jnp.int32),
            pltpu.VMEM((GRP, CHUNK), jnp.int32),
            pltpu.VMEM((CHUNK, D), jnp.float32),
            pltpu.VMEM((CHUNK, D), jnp.float32),
            pltpu.VMEM_SHARED((N, D), jnp.float32),
            pltpu.SemaphoreType.DMA,
            pltpu.SemaphoreType.DMA,
        ],
    )(_sc_agg_body)


def _sc_agg_body(x_hbm, src_hbm, dst_hbm, agg_out,
                 src_v, dst_v, rows_a, rows_b, acc_sh, sem_a, sem_b):
    c = lax.axis_index("c")
    s = lax.axis_index("s")
    wid = c * NS + s
    _fill_block(rows_a, 0.0)
    _zero_acc_slice(s, rows_a, acc_sh)
    plsc.subcore_barrier()

    def _group(g, carry):
        gb = wid * NGRP + g
        pltpu.sync_copy(src_hbm.at[gb], src_v)
        pltpu.sync_copy(dst_hbm.at[gb], dst_v)
        for b in range(0, GRP, 2):
            cpa = pltpu.async_copy(x_hbm.at[src_v.at[b]], rows_a, sem_a)
            cpb = pltpu.async_copy(x_hbm.at[src_v.at[b + 1]], rows_b, sem_b)
            cpa.wait()
            pltpu.sync_copy(rows_a, acc_sh.at[dst_v.at[b]], add=True)
            cpb.wait()
            pltpu.sync_copy(rows_b, acc_sh.at[dst_v.at[b + 1]], add=True)
        return carry
    lax.fori_loop(0, NGRP, _group, 0)

    plsc.subcore_barrier()
    _write_acc_slice(c, s, rows_a, acc_sh, agg_out)


@functools.cache
def _make_sc_deg():
    mesh = plsc.VectorSubcoreMesh(core_axis_name="c", subcore_axis_name="s",
                                  num_cores=NC)
    return functools.partial(
        pl.kernel,
        out_type=jax.ShapeDtypeStruct((NC * N, D), jnp.float32),
        mesh=mesh,
        scratch_types=[
            pltpu.VMEM((GRP, CHUNK), jnp.int32),
            pltpu.VMEM((CHUNK, D), jnp.float32),
            pltpu.VMEM_SHARED((N, D), jnp.float32),
        ],
    )(_sc_deg_body)


def _sc_deg_body(dst_hbm, deg_out, dst_v, ones_v, deg_sh):
    c = lax.axis_index("c")
    s = lax.axis_index("s")
    wid = c * NS + s
    _fill_block(ones_v, 0.0)
    _zero_acc_slice(s, ones_v, deg_sh)
    _fill_block(ones_v, 1.0)
    plsc.subcore_barrier()

    def _group(g, carry):
        gb = wid * NGRP + g
        pltpu.sync_copy(dst_hbm.at[gb], dst_v)
        for b in range(GRP):
            pltpu.sync_copy(ones_v, deg_sh.at[dst_v.at[b]], add=True)
        return carry
    lax.fori_loop(0, NGRP, _group, 0)

    plsc.subcore_barrier()
    _write_acc_slice(c, s, ones_v, deg_sh, deg_out)


BLK = 2000


def _tc_body(x_ref, a_ref, d_ref, wm_ref, bm_ref, ws_ref, bs_ref,
             g_ref, b_ref, o_ref):
    x = x_ref[...]
    a = a_ref[0] + a_ref[1]
    deg = d_ref[0, :, 0:1] + d_ref[1, :, 0:1]
    pid = pl.program_id(0)
    row0 = (lax.broadcasted_iota(jnp.int32, (BLK, 1), 0) == 0).astype(jnp.float32)
    row0 = row0 * (pid == 0).astype(jnp.float32)
    a = a - row0 * (PADC * x[0:1, :])
    deg = deg - row0 * PADC
    h = lax.dot_general(a, wm_ref[...], (((1,), (1,)), ((), ())),
                        precision=lax.Precision.HIGHEST,
                        preferred_element_type=jnp.float32)
    agg = (h + deg * bm_ref[...]) / jnp.maximum(deg, 1.0)
    o = lax.dot_general(x, ws_ref[...], (((1,), (1,)), ((), ())),
                        precision=lax.Precision.HIGHEST,
                        preferred_element_type=jnp.float32)
    o = o + bs_ref[...] + agg
    u = x + 0.5 * o * (1.0 + lax.erf(o * 0.7071067811865475))
    mu = jnp.mean(u, axis=1, keepdims=True)
    uc = u - mu
    var = jnp.mean(uc * uc, axis=1, keepdims=True)
    o_ref[...] = uc * lax.rsqrt(var + 1e-5) * g_ref[...] + b_ref[...]


def kernel(x, edge_index, Wm, bm, Ws, bs, gamma, beta):
    src = edge_index[0].astype(jnp.int32)
    dst = edge_index[1].astype(jnp.int32)
    pad = E_PAD - E
    src = jnp.concatenate([src, jnp.zeros((pad,), jnp.int32)])
    dst = jnp.concatenate([dst, jnp.zeros((pad,), jnp.int32)])
    src2 = src.reshape(NW * NGRP, GRP, CHUNK)
    dst2 = dst.reshape(NW * NGRP, GRP, CHUNK)

    agg = _make_sc_agg()(x, src2, dst2).reshape(NC, N, D)
    deg = _make_sc_deg()(dst2).reshape(NC, N, D)

    out = pl.pallas_call(
        _tc_body,
        grid=(N // BLK,),
        in_specs=[
            pl.BlockSpec((BLK, D), lambda i: (i, 0)),
            pl.BlockSpec((NC, BLK, D), lambda i: (0, i, 0)),
            pl.BlockSpec((NC, BLK, D), lambda i: (0, i, 0)),
            pl.BlockSpec((D, D), lambda i: (0, 0)),
            pl.BlockSpec((1, D), lambda i: (0, 0)),
            pl.BlockSpec((D, D), lambda i: (0, 0)),
            pl.BlockSpec((1, D), lambda i: (0, 0)),
            pl.BlockSpec((1, D), lambda i: (0, 0)),
            pl.BlockSpec((1, D), lambda i: (0, 0)),
        ],
        out_specs=pl.BlockSpec((BLK, D), lambda i: (i, 0)),
        out_shape=jax.ShapeDtypeStruct((N, D), jnp.float32),
    )(x, agg, deg, Wm, bm.reshape(1, D), Ws, bs.reshape(1, D),
      gamma.reshape(1, D), beta.reshape(1, D))
    return out


# speedup vs baseline: 3.5991x; 1.0793x over previous
import functools

import jax
import jax.numpy as jnp
from jax import lax
from jax.experimental import pallas as pl
from jax.experimental.pallas import tpu as pltpu
from jax.experimental.pallas import tpu_sc as plsc

N = 10000
E = 320000
D = 128

NC = 2
NS = 16
NW = NC * NS
CHUNK = 128
GRP = 8
NGRP = -(-E // (NW * CHUNK * GRP))
NCHUNK = NGRP * GRP
E_PAD = NW * NCHUNK * CHUNK
PADC = float(E_PAD - E)
RPT = 632
RPT_LAST = N - (NS - 1) * RPT
NTA = RPT - (RPT // CHUNK) * CHUNK
NTB = RPT_LAST - (RPT // CHUNK) * CHUNK


def _fill_block(ref, val):
    def _row(r, carry):
        for k in range(D // 16):
            ref[r, pl.ds(k * 16, 16)] = jnp.full((16,), val, jnp.float32)
        return carry
    lax.fori_loop(0, CHUNK, _row, 0)


def _zero_acc_slice(s, buf, acc_sh):
    base = s * RPT
    for k in range(RPT // CHUNK):
        pltpu.sync_copy(buf, acc_sh.at[pl.ds(base + k * CHUNK, CHUNK)])
    toff = base + (RPT // CHUNK) * CHUNK

    @pl.when(s < NS - 1)
    def _():
        pltpu.sync_copy(buf.at[pl.ds(0, NTA)], acc_sh.at[pl.ds(toff, NTA)])

    @pl.when(s == NS - 1)
    def _():
        pltpu.sync_copy(buf.at[pl.ds(0, NTB)], acc_sh.at[pl.ds(toff, NTB)])


def _write_acc_slice(c, s, buf, acc_sh, out_hbm):
    base = s * RPT
    hb = c * N
    for k in range(RPT // CHUNK):
        off = base + k * CHUNK
        pltpu.sync_copy(acc_sh.at[pl.ds(off, CHUNK)], buf)
        pltpu.sync_copy(buf, out_hbm.at[pl.ds(hb + off, CHUNK)])
    toff = base + (RPT // CHUNK) * CHUNK

    @pl.when(s < NS - 1)
    def _():
        pltpu.sync_copy(acc_sh.at[pl.ds(toff, NTA)], buf.at[pl.ds(0, NTA)])
        pltpu.sync_copy(buf.at[pl.ds(0, NTA)], out_hbm.at[pl.ds(hb + toff, NTA)])

    @pl.when(s == NS - 1)
    def _():
        pltpu.sync_copy(acc_sh.at[pl.ds(toff, NTB)], buf.at[pl.ds(0, NTB)])
        pltpu.sync_copy(buf.at[pl.ds(0, NTB)], out_hbm.at[pl.ds(hb + toff, NTB)])


@functools.cache
def _make_sc_agg():
    mesh = plsc.VectorSubcoreMesh(core_axis_name="c", subcore_axis_name="s",
                                  num_cores=NC)
    return functools.partial(
        pl.kernel,
        out_type=jax.ShapeDtypeStruct((NC * N, D), jnp.float32),
        mesh=mesh,
        scratch_types=[
            pltpu.VMEM((GRP, CHUNK), jnp.int32),
            pltpu.VMEM((GRP, CHUNK), jnp.int32),
            pltpu.VMEM((CHUNK, D), jnp.float32),
            pltpu.VMEM((CHUNK, D), jnp.float32),
            pltpu.VMEM_SHARED((N, D), jnp.float32),
            pltpu.SemaphoreType.DMA,
            pltpu.SemaphoreType.DMA,
        ],
    )(_sc_agg_body)


def _sc_agg_body(x_hbm, src_hbm, dst_hbm, agg_out,
                 src_v, dst_v, rows_a, rows_b, acc_sh, sem_a, sem_b):
    c = lax.axis_index("c")
    s = lax.axis_index("s")
    wid = c * NS + s
    _fill_block(rows_a, 0.0)
    _zero_acc_slice(s, rows_a, acc_sh)
    plsc.subcore_barrier()

    bufs = ((rows_a, sem_a), (rows_b, sem_b))

    def _group(g, carry):
        gb = wid * NGRP + g
        pltpu.sync_copy(src_hbm.at[gb], src_v)
        pltpu.sync_copy(dst_hbm.at[gb], dst_v)
        cps = [None] * GRP
        cps[0] = pltpu.async_copy(x_hbm.at[src_v.at[0]], rows_a, sem_a)
        for b in range(GRP):
            if b + 1 < GRP:
                nxt, nsem = bufs[(b + 1) % 2]
                cps[b + 1] = pltpu.async_copy(x_hbm.at[src_v.at[b + 1]], nxt, nsem)
            cps[b].wait()
            pltpu.sync_copy(bufs[b % 2][0], acc_sh.at[dst_v.at[b]], add=True)
        return carry
    lax.fori_loop(0, NGRP, _group, 0)

    plsc.subcore_barrier()
    _write_acc_slice(c, s, rows_a, acc_sh, agg_out)


@functools.cache
def _make_sc_deg():
    mesh = plsc.VectorSubcoreMesh(core_axis_name="c", subcore_axis_name="s",
                                  num_cores=NC)
    return functools.partial(
        pl.kernel,
        out_type=jax.ShapeDtypeStruct((NC * N, D), jnp.float32),
        mesh=mesh,
        scratch_types=[
            pltpu.VMEM((GRP, CHUNK), jnp.int32),
            pltpu.VMEM((CHUNK, D), jnp.float32),
            pltpu.VMEM_SHARED((N, D), jnp.float32),
        ],
    )(_sc_deg_body)


def _sc_deg_body(dst_hbm, deg_out, dst_v, ones_v, deg_sh):
    c = lax.axis_index("c")
    s = lax.axis_index("s")
    wid = c * NS + s
    _fill_block(ones_v, 0.0)
    _zero_acc_slice(s, ones_v, deg_sh)
    _fill_block(ones_v, 1.0)
    plsc.subcore_barrier()

    def _group(g, carry):
        gb = wid * NGRP + g
        pltpu.sync_copy(dst_hbm.at[gb], dst_v)
        for b in range(GRP):
            pltpu.sync_copy(ones_v, deg_sh.at[dst_v.at[b]], add=True)
        return carry
    lax.fori_loop(0, NGRP, _group, 0)

    plsc.subcore_barrier()
    _write_acc_slice(c, s, ones_v, deg_sh, deg_out)


BLK = 2000


def _tc_body(x_ref, a_ref, d_ref, wm_ref, bm_ref, ws_ref, bs_ref,
             g_ref, b_ref, o_ref):
    x = x_ref[...]
    a = a_ref[0] + a_ref[1]
    deg = d_ref[0, :, 0:1] + d_ref[1, :, 0:1]
    pid = pl.program_id(0)
    row0 = (lax.broadcasted_iota(jnp.int32, (BLK, 1), 0) == 0).astype(jnp.float32)
    row0 = row0 * (pid == 0).astype(jnp.float32)
    a = a - row0 * (PADC * x[0:1, :])
    deg = deg - row0 * PADC
    h = lax.dot_general(a, wm_ref[...], (((1,), (1,)), ((), ())),
                        precision=lax.Precision.HIGHEST,
                        preferred_element_type=jnp.float32)
    agg = (h + deg * bm_ref[...]) / jnp.maximum(deg, 1.0)
    o = lax.dot_general(x, ws_ref[...], (((1,), (1,)), ((), ())),
                        precision=lax.Precision.HIGHEST,
                        preferred_element_type=jnp.float32)
    o = o + bs_ref[...] + agg
    u = x + 0.5 * o * (1.0 + lax.erf(o * 0.7071067811865475))
    mu = jnp.mean(u, axis=1, keepdims=True)
    uc = u - mu
    var = jnp.mean(uc * uc, axis=1, keepdims=True)
    o_ref[...] = uc * lax.rsqrt(var + 1e-5) * g_ref[...] + b_ref[...]


def kernel(x, edge_index, Wm, bm, Ws, bs, gamma, beta):
    src = edge_index[0].astype(jnp.int32)
    dst = edge_index[1].astype(jnp.int32)
    pad = E_PAD - E
    src = jnp.concatenate([src, jnp.zeros((pad,), jnp.int32)])
    dst = jnp.concatenate([dst, jnp.zeros((pad,), jnp.int32)])
    src2 = src.reshape(NW * NGRP, GRP, CHUNK)
    dst2 = dst.reshape(NW * NGRP, GRP, CHUNK)

    agg = _make_sc_agg()(x, src2, dst2).reshape(NC, N, D)
    deg = _make_sc_deg()(dst2).reshape(NC, N, D)

    out = pl.pallas_call(
        _tc_body,
        grid=(N // BLK,),
        in_specs=[
            pl.BlockSpec((BLK, D), lambda i: (i, 0)),
            pl.BlockSpec((NC, BLK, D), lambda i: (0, i, 0)),
            pl.BlockSpec((NC, BLK, D), lambda i: (0, i, 0)),
            pl.BlockSpec((D, D), lambda i: (0, 0)),
            pl.BlockSpec((1, D), lambda i: (0, 0)),
            pl.BlockSpec((D, D), lambda i: (0, 0)),
            pl.BlockSpec((1, D), lambda i: (0, 0)),
            pl.BlockSpec((1, D), lambda i: (0, 0)),
            pl.BlockSpec((1, D), lambda i: (0, 0)),
        ],
        out_specs=pl.BlockSpec((BLK, D), lambda i: (i, 0)),
        out_shape=jax.ShapeDtypeStruct((N, D), jnp.float32),
    )(x, agg, deg, Wm, bm.reshape(1, D), Ws, bs.reshape(1, D),
      gamma.reshape(1, D), beta.reshape(1, D))
    return out


# speedup vs baseline: 4.1398x; 1.1502x over previous
import functools

import jax
import jax.numpy as jnp
from jax import lax
from jax.experimental import pallas as pl
from jax.experimental.pallas import tpu as pltpu
from jax.experimental.pallas import tpu_sc as plsc

N = 10000
E = 320000
D = 128

NC = 2
NS = 16
NW = NC * NS
CHUNK = 128
GRP = 8
NGRP = -(-E // (NW * CHUNK * GRP))
NCHUNK = NGRP * GRP
E_PAD = NW * NCHUNK * CHUNK
PADC = float(E_PAD - E)
RPT = 632
RPT_LAST = N - (NS - 1) * RPT
NTA = RPT - (RPT // CHUNK) * CHUNK
NTB = RPT_LAST - (RPT // CHUNK) * CHUNK


def _fill_block(ref, val):
    def _row(r, carry):
        for k in range(D // 16):
            ref[r, pl.ds(k * 16, 16)] = jnp.full((16,), val, jnp.float32)
        return carry
    lax.fori_loop(0, CHUNK, _row, 0)


def _zero_acc_slice(s, buf, acc_sh):
    base = s * RPT
    for k in range(RPT // CHUNK):
        pltpu.sync_copy(buf, acc_sh.at[pl.ds(base + k * CHUNK, CHUNK)])
    toff = base + (RPT // CHUNK) * CHUNK

    @pl.when(s < NS - 1)
    def _():
        pltpu.sync_copy(buf.at[pl.ds(0, NTA)], acc_sh.at[pl.ds(toff, NTA)])

    @pl.when(s == NS - 1)
    def _():
        pltpu.sync_copy(buf.at[pl.ds(0, NTB)], acc_sh.at[pl.ds(toff, NTB)])


def _write_acc_slice(c, s, buf, acc_sh, out_hbm):
    base = s * RPT
    hb = c * N
    for k in range(RPT // CHUNK):
        off = base + k * CHUNK
        pltpu.sync_copy(acc_sh.at[pl.ds(off, CHUNK)], buf)
        pltpu.sync_copy(buf, out_hbm.at[pl.ds(hb + off, CHUNK)])
    toff = base + (RPT // CHUNK) * CHUNK

    @pl.when(s < NS - 1)
    def _():
        pltpu.sync_copy(acc_sh.at[pl.ds(toff, NTA)], buf.at[pl.ds(0, NTA)])
        pltpu.sync_copy(buf.at[pl.ds(0, NTA)], out_hbm.at[pl.ds(hb + toff, NTA)])

    @pl.when(s == NS - 1)
    def _():
        pltpu.sync_copy(acc_sh.at[pl.ds(toff, NTB)], buf.at[pl.ds(0, NTB)])
        pltpu.sync_copy(buf.at[pl.ds(0, NTB)], out_hbm.at[pl.ds(hb + toff, NTB)])


@functools.cache
def _make_sc_agg():
    mesh = plsc.VectorSubcoreMesh(core_axis_name="c", subcore_axis_name="s",
                                  num_cores=NC)
    return functools.partial(
        pl.kernel,
        out_type=jax.ShapeDtypeStruct((NC * N, D), jnp.float32),
        mesh=mesh,
        scratch_types=[
            pltpu.VMEM((GRP, CHUNK), jnp.int32),
            pltpu.VMEM((GRP, CHUNK), jnp.int32),
            pltpu.VMEM((CHUNK, D), jnp.float32),
            pltpu.VMEM((CHUNK, D), jnp.float32),
            pltpu.VMEM_SHARED((N, D), jnp.float32),
            pltpu.SemaphoreType.DMA,
            pltpu.SemaphoreType.DMA,
        ],
    )(_sc_agg_body)


def _sc_agg_body(x_hbm, src_hbm, dst_hbm, agg_out,
                 src_v, dst_v, rows_a, rows_b, acc_sh, sem_a, sem_b):
    c = lax.axis_index("c")
    s = lax.axis_index("s")
    wid = c * NS + s
    _fill_block(rows_a, 0.0)
    _zero_acc_slice(s, rows_a, acc_sh)
    plsc.subcore_barrier()

    bufs = ((rows_a, sem_a), (rows_b, sem_b))

    def _group(g, carry):
        gb = wid * NGRP + g
        pltpu.sync_copy(src_hbm.at[gb], src_v)
        pltpu.sync_copy(dst_hbm.at[gb], dst_v)
        cps = [None] * GRP
        cps[0] = pltpu.async_copy(x_hbm.at[src_v.at[0]], rows_a, sem_a)
        for b in range(GRP):
            if b + 1 < GRP:
                nxt, nsem = bufs[(b + 1) % 2]
                cps[b + 1] = pltpu.async_copy(x_hbm.at[src_v.at[b + 1]], nxt, nsem)
            cps[b].wait()
            pltpu.sync_copy(bufs[b % 2][0], acc_sh.at[dst_v.at[b]], add=True)
        return carry
    lax.fori_loop(0, NGRP, _group, 0)

    plsc.subcore_barrier()
    _write_acc_slice(c, s, rows_a, acc_sh, agg_out)


@functools.cache
def _make_sc_deg():
    mesh = plsc.VectorSubcoreMesh(core_axis_name="c", subcore_axis_name="s",
                                  num_cores=NC)
    return functools.partial(
        pl.kernel,
        out_type=jax.ShapeDtypeStruct((NC * N, D), jnp.float32),
        mesh=mesh,
        scratch_types=[
            pltpu.VMEM((GRP, CHUNK), jnp.int32),
            pltpu.VMEM((CHUNK, D), jnp.float32),
            pltpu.VMEM_SHARED((N, D), jnp.float32),
        ],
    )(_sc_deg_body)


def _sc_deg_body(dst_hbm, deg_out, dst_v, ones_v, deg_sh):
    c = lax.axis_index("c")
    s = lax.axis_index("s")
    wid = c * NS + s
    _fill_block(ones_v, 0.0)
    _zero_acc_slice(s, ones_v, deg_sh)
    _fill_block(ones_v, 1.0)
    plsc.subcore_barrier()

    def _group(g, carry):
        gb = wid * NGRP + g
        pltpu.sync_copy(dst_hbm.at[gb], dst_v)
        for b in range(GRP):
            pltpu.sync_copy(ones_v, deg_sh.at[dst_v.at[b]], add=True)
        return carry
    lax.fori_loop(0, NGRP, _group, 0)

    plsc.subcore_barrier()
    _write_acc_slice(c, s, ones_v, deg_sh, deg_out)


BLK = 2000


def _tc_body(x_ref, a_ref, d_ref, wm_ref, bm_ref, ws_ref, bs_ref,
             g_ref, b_ref, o_ref):
    x = x_ref[...]
    a = a_ref[0] + a_ref[1]
    deg = d_ref[0, :, 0:1] + d_ref[1, :, 0:1]
    pid = pl.program_id(0)
    row0 = (lax.broadcasted_iota(jnp.int32, (BLK, 1), 0) == 0).astype(jnp.float32)
    row0 = row0 * (pid == 0).astype(jnp.float32)
    a = a - row0 * (PADC * x[0:1, :])
    deg = deg - row0 * PADC
    h = lax.dot_general(a, wm_ref[...], (((1,), (1,)), ((), ())),
                        precision=lax.Precision.HIGHEST,
                        preferred_element_type=jnp.float32)
    agg = (h + deg * bm_ref[...]) / jnp.maximum(deg, 1.0)
    o = lax.dot_general(x, ws_ref[...], (((1,), (1,)), ((), ())),
                        precision=lax.Precision.HIGHEST,
                        preferred_element_type=jnp.float32)
    o = o + bs_ref[...] + agg
    u = x + 0.5 * o * (1.0 + lax.erf(o * 0.7071067811865475))
    mu = jnp.mean(u, axis=1, keepdims=True)
    uc = u - mu
    var = jnp.mean(uc * uc, axis=1, keepdims=True)
    o_ref[...] = uc * lax.rsqrt(var + 1e-5) * g_ref[...] + b_ref[...]


def kernel(x, edge_index, Wm, bm, Ws, bs, gamma, beta):
    src = edge_index[0].astype(jnp.int32)
    dst = edge_index[1].astype(jnp.int32)
    pad = E_PAD - E
    src = jnp.concatenate([src, jnp.zeros((pad,), jnp.int32)])
    dst = jnp.concatenate([dst, jnp.zeros((pad,), jnp.int32)])
    src2 = src.reshape(NW, NGRP, GRP, CHUNK)
    src2 = src2 + (jnp.arange(NW, dtype=jnp.int32) // NS * N)[:, None, None, None]
    src2 = src2.reshape(NW * NGRP, GRP, CHUNK)
    dst2 = dst.reshape(NW * NGRP, GRP, CHUNK)
    xx = jnp.concatenate([x, x], axis=0)

    agg = _make_sc_agg()(xx, src2, dst2).reshape(NC, N, D)
    deg = _make_sc_deg()(dst2).reshape(NC, N, D)

    out = pl.pallas_call(
        _tc_body,
        grid=(N // BLK,),
        in_specs=[
            pl.BlockSpec((BLK, D), lambda i: (i, 0)),
            pl.BlockSpec((NC, BLK, D), lambda i: (0, i, 0)),
            pl.BlockSpec((NC, BLK, D), lambda i: (0, i, 0)),
            pl.BlockSpec((D, D), lambda i: (0, 0)),
            pl.BlockSpec((1, D), lambda i: (0, 0)),
            pl.BlockSpec((D, D), lambda i: (0, 0)),
            pl.BlockSpec((1, D), lambda i: (0, 0)),
            pl.BlockSpec((1, D), lambda i: (0, 0)),
            pl.BlockSpec((1, D), lambda i: (0, 0)),
        ],
        out_specs=pl.BlockSpec((BLK, D), lambda i: (i, 0)),
        out_shape=jax.ShapeDtypeStruct((N, D), jnp.float32),
    )(x, agg, deg, Wm, bm.reshape(1, D), Ws, bs.reshape(1, D),
      gamma.reshape(1, D), beta.reshape(1, D))
    return out
